# baseline (speedup 1.0000x reference)
import functools

import jax
import jax.numpy as jnp
from jax import lax
from jax.experimental import pallas as pl
from jax.experimental.pallas import tpu as pltpu
from jax.experimental.pallas import tpu_sc as plsc

EMB_DIM = 32
BATCH = 4096
HIST = 50
TOTAL = BATCH * HIST
NUM_CORES = 2
NUM_SUBCORES = 16
NW = NUM_CORES * NUM_SUBCORES
PER_W = TOTAL // NW
CHUNK = 128
NCH = PER_W // CHUNK

_mesh = plsc.VectorSubcoreMesh(core_axis_name="c", subcore_axis_name="s")


@functools.partial(
    pl.kernel,
    mesh=_mesh,
    out_type=[jax.ShapeDtypeStruct((TOTAL, EMB_DIM), jnp.float32)] * 3,
    scratch_types=[
        pltpu.VMEM((NCH, CHUNK), jnp.int32),
        pltpu.VMEM((CHUNK, EMB_DIM), jnp.float32),
        pltpu.SemaphoreType.DMA,
    ],
    compiler_params=pltpu.CompilerParams(use_tc_tiling_on_sc=False),
)
def _gather3(qry_hbm, pos_hbm, neg_hbm, table_hbm,
             out_q, out_p, out_n, idx_v, rows_v, sem):
    wid = lax.axis_index("s") * NUM_CORES + lax.axis_index("c")
    base = wid * PER_W

    for idx_hbm, out_hbm in ((qry_hbm, out_q), (pos_hbm, out_p),
                             (neg_hbm, out_n)):
        pltpu.sync_copy(idx_hbm.at[wid], idx_v)

        def body(j, _):
            pltpu.async_copy(table_hbm.at[idx_v.at[j]], rows_v, sem).wait()
            pltpu.sync_copy(rows_v, out_hbm.at[pl.ds(base + j * CHUNK, CHUNK)])
            return ()

        lax.fori_loop(0, NCH, body, ())


def kernel(qry_lkup, pos_lkup, neg_lkup, table):
    shaped = lambda a: a.astype(jnp.int32).reshape(NW, NCH, CHUNK)
    out_q, out_p, out_n = _gather3(
        shaped(qry_lkup), shaped(pos_lkup), shaped(neg_lkup), table)
    out_shape = (BATCH, HIST, EMB_DIM)
    return (out_q.reshape(out_shape), out_p.reshape(out_shape),
            out_n.reshape(out_shape))
